# baseline (speedup 1.0000x reference)
import functools

import jax
import jax.numpy as jnp
from jax import lax
from jax.experimental import pallas as pl
from jax.experimental.pallas import tpu as pltpu
from jax.experimental.pallas import tpu_sc as plsc

N = 10000
E = 320000
D_IN = 128
H1 = 64
H2 = 32

NC = 2
NS = 16
NW = NC * NS
E_PER_W = E // NW
E_PER_W_PAD = 10240
N_PAD = 10240
ROWS_PER_TILE = N_PAD // NS
CHUNK1 = 512
NCHUNK1 = E_PER_W_PAD // CHUNK1
CHUNK2 = 1024
NCHUNK2 = E_PER_W_PAD // CHUNK2


def _edge_pipeline(table, srcs, dsts, acc, si0, si1, di0, di1, r0, r1,
                   gs0, gs1, ss0, ss1, w, nchunk):
    def body(s, carry):
        i0 = 2 * s
        i1 = i0 + 1

        @pl.when(s > 0)
        def _():
            pltpu.make_async_copy(r0, acc.at[di0], ss0).wait()
            pltpu.make_async_copy(r1, acc.at[di1], ss1).wait()

        pltpu.sync_copy(srcs.at[w, i0], si0)
        pltpu.sync_copy(dsts.at[w, i0], di0)
        g0 = pltpu.async_copy(table.at[si0], r0, gs0)
        pltpu.sync_copy(srcs.at[w, i1], si1)
        pltpu.sync_copy(dsts.at[w, i1], di1)
        g1 = pltpu.async_copy(table.at[si1], r1, gs1)
        g0.wait()
        pltpu.async_copy(r0, acc.at[di0], ss0, add=True)
        g1.wait()
        pltpu.async_copy(r1, acc.at[di1], ss1, add=True)
        return carry

    lax.fori_loop(0, nchunk // 2, body, 0)
    pltpu.make_async_copy(r0, acc.at[di0], ss0).wait()
    pltpu.make_async_copy(r1, acc.at[di1], ss1).wait()


def _sc_layer_call(tables, srcs, dsts, zeros, d, chunk, nchunk):
    mesh = plsc.VectorSubcoreMesh(core_axis_name="c", subcore_axis_name="s")
    osh = jax.ShapeDtypeStruct((NC, N_PAD, d), jnp.float32)

    @functools.partial(
        pl.kernel,
        mesh=mesh,
        compiler_params=pltpu.CompilerParams(use_tc_tiling_on_sc=False),
        out_type=[osh, osh, osh, osh],
        scratch_types=[
            pltpu.VMEM((chunk,), jnp.int32),
            pltpu.VMEM((chunk,), jnp.int32),
            pltpu.VMEM((chunk,), jnp.int32),
            pltpu.VMEM((chunk,), jnp.int32),
            pltpu.VMEM((chunk, d), jnp.float32),
            pltpu.VMEM((chunk, d), jnp.float32),
            pltpu.VMEM_SHARED((N_PAD, d), jnp.float32),
            pltpu.SemaphoreType.DMA,
            pltpu.SemaphoreType.DMA,
            pltpu.SemaphoreType.DMA,
            pltpu.SemaphoreType.DMA,
        ],
    )
    def k(t0, t1, t2, t3, s0, s1, s2, s3, d0, d1, d2, d3, zeros_hbm,
          o0, o1, o2, o3,
          si0, si1, di0, di1, r0, r1, acc, gs0, gs1, ss0, ss1):
        c = lax.axis_index("c")
        s = lax.axis_index("s")
        w = c * NS + s
        row0 = s * ROWS_PER_TILE
        rows = pl.ds(row0, ROWS_PER_TILE)
        pltpu.sync_copy(zeros_hbm.at[rows], acc.at[rows])
        plsc.subcore_barrier()
        for t, (tab, sr, ds_, out) in enumerate(
                zip((t0, t1, t2, t3), (s0, s1, s2, s3),
                    (d0, d1, d2, d3), (o0, o1, o2, o3))):
            _edge_pipeline(tab, sr, ds_, acc, si0, si1, di0, di1, r0, r1,
                           gs0, gs1, ss0, ss1, w, nchunk)
            plsc.subcore_barrier()
            pltpu.sync_copy(acc.at[rows], out.at[c, rows])
            if t < 3:
                pltpu.sync_copy(zeros_hbm.at[rows], acc.at[rows])
            plsc.subcore_barrier()

    return k(*tables, *srcs, *dsts, zeros)


def _edges_padded(ei):
    ei = ei.astype(jnp.int32)
    src = ei[1].reshape(NW, E_PER_W)
    dst = ei[0].reshape(NW, E_PER_W)
    pad = E_PER_W_PAD - E_PER_W
    src = jnp.pad(src, ((0, 0), (0, pad)), constant_values=0)
    dst = jnp.pad(dst, ((0, 0), (0, pad)), constant_values=N)
    return src, dst


def _l2n(x):
    n = jnp.sqrt(jnp.maximum(jnp.sum(x * x, axis=1, keepdims=True), 1e-12))
    return x / n


_RB = 1000


def _t1_body(f0, f1, w00, w01, w10, w11, h00, h01, h10, h11):
    a = f0[...]
    b = f1[...]
    h00[...] = jnp.dot(a, w00[...], preferred_element_type=jnp.float32)
    h01[...] = jnp.dot(b, w01[...], preferred_element_type=jnp.float32)
    h10[...] = jnp.dot(a, w10[...], preferred_element_type=jnp.float32)
    h11[...] = jnp.dot(b, w11[...], preferred_element_type=jnp.float32)


def _t1(f0, f1, w00, w01, w10, w11):
    fs = pl.BlockSpec((_RB, D_IN), lambda i: (i, 0))
    ws = pl.BlockSpec((D_IN, H1), lambda i: (0, 0))
    os = pl.BlockSpec((_RB, H1), lambda i: (i, 0))
    sh = jax.ShapeDtypeStruct((N, H1), jnp.float32)
    return pl.pallas_call(
        _t1_body,
        grid=(N // _RB,),
        in_specs=[fs, fs, ws, ws, ws, ws],
        out_specs=[os, os, os, os],
        out_shape=[sh, sh, sh, sh],
    )(f0, f1, w00, w01, w10, w11)


def _t2_body(a00, a01, a10, a11, w00, w01, w10, w11, g00, g01, g10, g11):
    h0 = jax.nn.relu(_l2n(a00[0] + a00[1]) + _l2n(a01[0] + a01[1]))
    h1 = jax.nn.relu(_l2n(a10[0] + a10[1]) + _l2n(a11[0] + a11[1]))
    g00[...] = jnp.dot(h0, w00[...], preferred_element_type=jnp.float32)
    g01[...] = jnp.dot(h1, w01[...], preferred_element_type=jnp.float32)
    g10[...] = jnp.dot(h0, w10[...], preferred_element_type=jnp.float32)
    g11[...] = jnp.dot(h1, w11[...], preferred_element_type=jnp.float32)


def _t2(a00, a01, a10, a11, w00, w01, w10, w11):
    asp = pl.BlockSpec((NC, _RB, H1), lambda i: (0, i, 0))
    ws = pl.BlockSpec((H1, H2), lambda i: (0, 0))
    os = pl.BlockSpec((_RB, H2), lambda i: (i, 0))
    sh = jax.ShapeDtypeStruct((N, H2), jnp.float32)
    return pl.pallas_call(
        _t2_body,
        grid=(N // _RB,),
        in_specs=[asp, asp, asp, asp, ws, ws, ws, ws],
        out_specs=[os, os, os, os],
        out_shape=[sh, sh, sh, sh],
    )(a00, a01, a10, a11, w00, w01, w10, w11)


def _t3_body(b00, b01, b10, b11, e0, e1):
    e0[...] = _l2n(b00[0] + b00[1]) + _l2n(b01[0] + b01[1])
    e1[...] = _l2n(b10[0] + b10[1]) + _l2n(b11[0] + b11[1])


def _t3(b00, b01, b10, b11):
    bsp = pl.BlockSpec((NC, _RB, H2), lambda i: (0, i, 0))
    os = pl.BlockSpec((_RB, H2), lambda i: (i, 0))
    sh = jax.ShapeDtypeStruct((N, H2), jnp.float32)
    return pl.pallas_call(
        _t3_body,
        grid=(N // _RB,),
        in_specs=[bsp, bsp, bsp, bsp],
        out_specs=[os, os],
        out_shape=[sh, sh],
    )(b00, b01, b10, b11)


def kernel(feat_0, feat_1, ei_00, ei_01, ei_10, ei_11,
           W1_00, W1_01, W1_10, W1_11,
           W2_00, W2_01, W2_10, W2_11):
    s00, d00 = _edges_padded(ei_00)
    s01, d01 = _edges_padded(ei_01)
    s10, d10 = _edges_padded(ei_10)
    s11, d11 = _edges_padded(ei_11)
    sl1 = [x.reshape(NW, NCHUNK1, CHUNK1) for x in (s00, s01, s10, s11)]
    dl1 = [x.reshape(NW, NCHUNK1, CHUNK1) for x in (d00, d01, d10, d11)]
    sl2 = [x.reshape(NW, NCHUNK2, CHUNK2) for x in (s00, s01, s10, s11)]
    dl2 = [x.reshape(NW, NCHUNK2, CHUNK2) for x in (d00, d01, d10, d11)]
    z1 = jnp.zeros((N_PAD, H1), jnp.float32)
    z2 = jnp.zeros((N_PAD, H2), jnp.float32)

    h00, h01, h10, h11 = _t1(feat_0, feat_1, W1_00, W1_01, W1_10, W1_11)

    a00, a01, a10, a11 = _sc_layer_call(
        (h00, h01, h10, h11), sl1, dl1, z1, H1, CHUNK1, NCHUNK1)

    g00, g01, g10, g11 = _t2(a00, a01, a10, a11, W2_00, W2_01, W2_10, W2_11)

    b00, b01, b10, b11 = _sc_layer_call(
        (g00, g01, g10, g11), sl2, dl2, z2, H2, CHUNK2, NCHUNK2)

    e0, e1 = _t3(b00, b01, b10, b11)
    return jnp.concatenate([e0, e1], axis=0)


# speedup vs baseline: 1.0023x; 1.0023x over previous
import functools

import jax
import jax.numpy as jnp
from jax import lax
from jax.experimental import pallas as pl
from jax.experimental.pallas import tpu as pltpu
from jax.experimental.pallas import tpu_sc as plsc

N = 10000
E = 320000
D_IN = 128
H1 = 64
H2 = 32

NC = 2
NS = 16
NW = NC * NS
E_PER_W = E // NW
E_PER_W_PAD = 10240
N_PAD = 10240
ROWS_PER_TILE = N_PAD // NS
CHUNK1 = 512
NCHUNK1 = E_PER_W_PAD // CHUNK1
CHUNK2 = 1280
NCHUNK2 = E_PER_W_PAD // CHUNK2


def _edge_pipeline(table, acc, idx_v, r0, r1, gs0, gs1, ss0, ss1, nchunk):
    def body(s, carry):
        i0 = 2 * s
        i1 = i0 + 1

        @pl.when(s > 0)
        def _():
            pltpu.make_async_copy(r0, acc.at[idx_v.at[1, i0]], ss0).wait()

        g0 = pltpu.async_copy(table.at[idx_v.at[0, i0]], r0, gs0)

        @pl.when(s > 0)
        def _():
            pltpu.make_async_copy(r1, acc.at[idx_v.at[1, i1]], ss1).wait()

        g1 = pltpu.async_copy(table.at[idx_v.at[0, i1]], r1, gs1)
        g0.wait()
        pltpu.async_copy(r0, acc.at[idx_v.at[1, i0]], ss0, add=True)
        g1.wait()
        pltpu.async_copy(r1, acc.at[idx_v.at[1, i1]], ss1, add=True)
        return carry

    lax.fori_loop(0, nchunk // 2, body, 0)
    pltpu.make_async_copy(r0, acc.at[idx_v.at[1, 0]], ss0).wait()
    pltpu.make_async_copy(r1, acc.at[idx_v.at[1, 1]], ss1).wait()


def _sc_layer_call(tables, idxs, zeros, d, chunk, nchunk):
    mesh = plsc.VectorSubcoreMesh(core_axis_name="c", subcore_axis_name="s")
    osh = jax.ShapeDtypeStruct((NC, N_PAD, d), jnp.float32)

    @functools.partial(
        pl.kernel,
        mesh=mesh,
        compiler_params=pltpu.CompilerParams(use_tc_tiling_on_sc=False),
        out_type=[osh, osh, osh, osh],
        scratch_types=[
            pltpu.VMEM((2, nchunk, chunk), jnp.int32),
            pltpu.VMEM((chunk, d), jnp.float32),
            pltpu.VMEM((chunk, d), jnp.float32),
            pltpu.VMEM_SHARED((N_PAD, d), jnp.float32),
            pltpu.SemaphoreType.DMA,
            pltpu.SemaphoreType.DMA,
            pltpu.SemaphoreType.DMA,
            pltpu.SemaphoreType.DMA,
        ],
    )
    def k(t0, t1, t2, t3, i0, i1, i2, i3, zeros_hbm,
          o0, o1, o2, o3,
          idx_v, r0, r1, acc, gs0, gs1, ss0, ss1):
        c = lax.axis_index("c")
        s = lax.axis_index("s")
        w = c * NS + s
        row0 = s * ROWS_PER_TILE
        rows = pl.ds(row0, ROWS_PER_TILE)
        pltpu.sync_copy(zeros_hbm.at[rows], acc.at[rows])
        plsc.subcore_barrier()
        for t, (tab, ih, out) in enumerate(
                zip((t0, t1, t2, t3), (i0, i1, i2, i3), (o0, o1, o2, o3))):
            pltpu.sync_copy(ih.at[w], idx_v)
            _edge_pipeline(tab, acc, idx_v, r0, r1, gs0, gs1, ss0, ss1,
                           nchunk)
            plsc.subcore_barrier()
            pltpu.sync_copy(acc.at[rows], out.at[c, rows])
            if t < 3:
                pltpu.sync_copy(zeros_hbm.at[rows], acc.at[rows])
            plsc.subcore_barrier()

    return k(*tables, *idxs, zeros)


def _edges_padded(ei):
    ei = ei.astype(jnp.int32)
    src = ei[1].reshape(NW, E_PER_W)
    dst = ei[0].reshape(NW, E_PER_W)
    pad = E_PER_W_PAD - E_PER_W
    src = jnp.pad(src, ((0, 0), (0, pad)), constant_values=0)
    dst = jnp.pad(dst, ((0, 0), (0, pad)), constant_values=N)
    return jnp.stack([src, dst], axis=1)


def _l2n(x):
    n = jnp.sqrt(jnp.maximum(jnp.sum(x * x, axis=1, keepdims=True), 1e-12))
    return x / n


_RB = 1000


def _t1_body(f0, f1, w00, w01, w10, w11, h00, h01, h10, h11):
    a = f0[...]
    b = f1[...]
    h00[...] = jnp.dot(a, w00[...], preferred_element_type=jnp.float32)
    h01[...] = jnp.dot(b, w01[...], preferred_element_type=jnp.float32)
    h10[...] = jnp.dot(a, w10[...], preferred_element_type=jnp.float32)
    h11[...] = jnp.dot(b, w11[...], preferred_element_type=jnp.float32)


def _t1(f0, f1, w00, w01, w10, w11):
    fs = pl.BlockSpec((_RB, D_IN), lambda i: (i, 0))
    ws = pl.BlockSpec((D_IN, H1), lambda i: (0, 0))
    os = pl.BlockSpec((_RB, H1), lambda i: (i, 0))
    sh = jax.ShapeDtypeStruct((N, H1), jnp.float32)
    return pl.pallas_call(
        _t1_body,
        grid=(N // _RB,),
        in_specs=[fs, fs, ws, ws, ws, ws],
        out_specs=[os, os, os, os],
        out_shape=[sh, sh, sh, sh],
    )(f0, f1, w00, w01, w10, w11)


def _t2_body(a00, a01, a10, a11, w00, w01, w10, w11, g00, g01, g10, g11):
    h0 = jax.nn.relu(_l2n(a00[0] + a00[1]) + _l2n(a01[0] + a01[1]))
    h1 = jax.nn.relu(_l2n(a10[0] + a10[1]) + _l2n(a11[0] + a11[1]))
    g00[...] = jnp.dot(h0, w00[...], preferred_element_type=jnp.float32)
    g01[...] = jnp.dot(h1, w01[...], preferred_element_type=jnp.float32)
    g10[...] = jnp.dot(h0, w10[...], preferred_element_type=jnp.float32)
    g11[...] = jnp.dot(h1, w11[...], preferred_element_type=jnp.float32)


def _t2(a00, a01, a10, a11, w00, w01, w10, w11):
    asp = pl.BlockSpec((NC, _RB, H1), lambda i: (0, i, 0))
    ws = pl.BlockSpec((H1, H2), lambda i: (0, 0))
    os = pl.BlockSpec((_RB, H2), lambda i: (i, 0))
    sh = jax.ShapeDtypeStruct((N, H2), jnp.float32)
    return pl.pallas_call(
        _t2_body,
        grid=(N // _RB,),
        in_specs=[asp, asp, asp, asp, ws, ws, ws, ws],
        out_specs=[os, os, os, os],
        out_shape=[sh, sh, sh, sh],
    )(a00, a01, a10, a11, w00, w01, w10, w11)


def _t3_body(b00, b01, b10, b11, e0, e1):
    e0[...] = _l2n(b00[0] + b00[1]) + _l2n(b01[0] + b01[1])
    e1[...] = _l2n(b10[0] + b10[1]) + _l2n(b11[0] + b11[1])


def _t3(b00, b01, b10, b11):
    bsp = pl.BlockSpec((NC, _RB, H2), lambda i: (0, i, 0))
    os = pl.BlockSpec((_RB, H2), lambda i: (i, 0))
    sh = jax.ShapeDtypeStruct((N, H2), jnp.float32)
    return pl.pallas_call(
        _t3_body,
        grid=(N // _RB,),
        in_specs=[bsp, bsp, bsp, bsp],
        out_specs=[os, os],
        out_shape=[sh, sh],
    )(b00, b01, b10, b11)


def kernel(feat_0, feat_1, ei_00, ei_01, ei_10, ei_11,
           W1_00, W1_01, W1_10, W1_11,
           W2_00, W2_01, W2_10, W2_11):
    packed = [_edges_padded(e) for e in (ei_00, ei_01, ei_10, ei_11)]
    il1 = [x.reshape(NW, 2, NCHUNK1, CHUNK1) for x in packed]
    il2 = [x.reshape(NW, 2, NCHUNK2, CHUNK2) for x in packed]
    z1 = jnp.zeros((N_PAD, H1), jnp.float32)
    z2 = jnp.zeros((N_PAD, H2), jnp.float32)

    h00, h01, h10, h11 = _t1(feat_0, feat_1, W1_00, W1_01, W1_10, W1_11)

    a00, a01, a10, a11 = _sc_layer_call(
        (h00, h01, h10, h11), il1, z1, H1, CHUNK1, NCHUNK1)

    g00, g01, g10, g11 = _t2(a00, a01, a10, a11, W2_00, W2_01, W2_10, W2_11)

    b00, b01, b10, b11 = _sc_layer_call(
        (g00, g01, g10, g11), il2, z2, H2, CHUNK2, NCHUNK2)

    e0, e1 = _t3(b00, b01, b10, b11)
    return jnp.concatenate([e0, e1], axis=0)
